# baseline (speedup 1.0000x reference)
import jax
import jax.numpy as jnp
import numpy as np
from jax.experimental import pallas as pl
from jax.experimental.pallas import tpu as pltpu

_NEG = -1e30
_K = 8


def _lrelu(x):
    return jnp.where(x >= 0, x, 0.2 * x)


def _dotT(a, b):
    return jax.lax.dot_general(a, b, (((0,), (0,)), ((), ())),
                               preferred_element_type=jnp.float32)


def _dotNT(a, b):
    return jax.lax.dot_general(a, b, (((1,), (1,)), ((), ())),
                               preferred_element_type=jnp.float32)


def _softmax_rows(score, adj_bool):
    masked = jnp.where(adj_bool, score, _NEG)
    m = jnp.max(masked, axis=1, keepdims=True)
    e = jnp.exp(masked - m)
    return e / jnp.sum(e, axis=1, keepdims=True)


def _egat_kernel(pni_ref, pei_ref,
                 H_ref, ECt_ref, ELt_ref, AH_ref, AEt_ref, MEt_ref, maskMM_ref,
                 WH_ref, WHb_ref, WEC_ref, WECb_ref, WEL_ref, WELb_ref,
                 aCr_ref, aLr_ref, bCr_ref, bLr_ref,
                 Hn_ref, ECn_ref, ELn_ref, Hm_ref,
                 rme_sc, inv_sc, invE_sc, dma_sem):
    N = 64
    M = 50
    D = 256
    B_ = 8
    Fe_ = 128
    b = pl.program_id(0)
    f32 = jnp.float32

    def _copies():
        for k in range(_K):
            icol = pni_ref[k] // 2
            yield pltpu.make_async_copy(
                MEt_ref.at[:, :, pl.ds(icol * 128, 128)],
                rme_sc.at[:, :, pl.ds(k * 128, 128)],
                dma_sem,
            )

    @pl.when(b == 0)
    def _():
        for c in _copies():
            c.start()

    H = H_ref[0]

    onehotB = (jax.lax.broadcasted_iota(jnp.int32, (B_, 1), 0) == b).astype(f32)
    selB = maskMM_ref[...] * jnp.broadcast_to(
        onehotB[None, :, :], (M, B_, 1)).reshape(M * B_, 1)
    ECall = ECt_ref[...].reshape(M * B_, Fe_)
    ELall = ELt_ref[...].reshape(M * B_, Fe_)
    EC = _dotT(selB, ECall)
    EL = _dotT(selB, ELall)
    aCr = aCr_ref[...]
    aLr = aLr_ref[...]
    bCr = bCr_ref[...]
    bLr = bLr_ref[...]
    WECb = WECb_ref[...].reshape(1, D)
    WELb = WELb_ref[...].reshape(1, D)

    Ht = jnp.dot(H, WH_ref[...], preferred_element_type=f32) + WHb_ref[...].reshape(1, D)
    FC = jnp.dot(EC, WEC_ref[...], preferred_element_type=f32)
    FL = jnp.dot(EL, WEL_ref[...], preferred_element_type=f32)

    wvC = _dotNT(WEC_ref[...], aCr[:, 2 * D:])
    wvL = _dotNT(WEL_ref[...], aLr[:, 2 * D:])
    fc = jnp.dot(EC, wvC, preferred_element_type=f32)
    fl = jnp.dot(EL, wvL, preferred_element_type=f32)
    fcfl = jnp.concatenate([fc, fl], axis=1)
    v_rowC = _dotNT(aCr[:, D:2 * D], Ht)
    v_rowL = _dotNT(aLr[:, D:2 * D], Ht)
    cstC = _dotNT(WECb, aCr[:, 2 * D:])
    cstL = _dotNT(WELb, aLr[:, 2 * D:])

    @pl.when(b == 0)
    def _():
        iota = jax.lax.broadcasted_iota(jnp.int32, (N, 1), 0)
        i_row = jnp.concatenate(
            [jnp.full((1, 1), pni_ref[k], jnp.int32) for k in range(_K)], axis=1)
        onehot0 = (iota == i_row).astype(f32)
        AHf = (AH_ref[0] > 0).astype(f32)
        adj80 = _dotT(onehot0, AHf)
        any_adj = jnp.max(AHf, axis=1, keepdims=True)
        any80 = _dotT(onehot0, any_adj)
        inv_sc[:, 0:8] = onehot0
        inv_sc[0:8, 8:8 + N] = adj80
        inv_sc[0:8, 8 + N:9 + N] = any80

        iotaE = jax.lax.broadcasted_iota(jnp.int32, (M, 1), 0)
        p_row = jnp.concatenate(
            [jnp.full((1, 1), pei_ref[k], jnp.int32) for k in range(_K)], axis=1)
        onehotE0 = (iotaE == p_row).astype(f32)
        AEf = (AEt_ref[:, 0, :] > 0).astype(f32)
        adjE80 = _dotT(onehotE0, AEf)
        anyE = jnp.max(AEf, axis=1, keepdims=True)
        anyE80 = _dotT(onehotE0, anyE)
        invE_sc[:, 0:8] = onehotE0
        invE_sc[0:8, 8:8 + M] = adjE80
        invE_sc[0:8, 8 + M:9 + M] = anyE80

    onehot = inv_sc[:, 0:8]
    adj8 = inv_sc[0:8, 8:8 + N]
    any8 = inv_sc[0:8, 8 + N:9 + N]
    rowsH8 = _dotT(onehot, Ht)
    u8C = _dotNT(rowsH8, aCr[:, :D])
    u8L = _dotNT(rowsH8, aLr[:, :D])

    ECt = FC + WECb
    ELt = FL + WELb
    y_rowC = _dotNT(bCr[:, D:2 * D], ECt)
    y_rowL = _dotNT(bLr[:, D:2 * D], ELt)
    zC = _dotNT(H[0:1, :], bCr[:, 2 * D:])
    zL = _dotNT(H[0:1, :], bLr[:, 2 * D:])

    onehotE = invE_sc[:, 0:8]
    adjE8 = invE_sc[0:8, 8:8 + M]
    anyE8 = invE_sc[0:8, 8 + M:9 + M]
    rowsEC8 = _dotT(onehotE, ECt)
    rowsEL8 = _dotT(onehotE, ELt)
    x8C = _dotNT(rowsEC8, bCr[:, :D])
    x8L = _dotNT(rowsEL8, bLr[:, :D])

    sc8EC = _lrelu(x8C + y_rowC + zC)
    sc8EL = _lrelu(x8L + y_rowL + zL)
    attn8EC = _softmax_rows(sc8EC, adjE8 > 0.5)
    attn8EL = _softmax_rows(sc8EL, adjE8 > 0.5)
    aggEC8 = jnp.dot(attn8EC, ECt, preferred_element_type=f32)
    aggEL8 = jnp.dot(attn8EL, ELt, preferred_element_type=f32)
    valEC = jnp.where(anyE8 > 0.5, aggEC8, rowsEC8)
    valEL = jnp.where(anyE8 > 0.5, aggEL8, rowsEL8)

    ECn_ref[0] = ECt
    ELn_ref[0] = ELt
    for k in range(_K):
        p = pei_ref[k]
        ECn_ref[0, pl.ds(p, 1), :] = valEC[k:k + 1, :]
        ELn_ref[0, pl.ds(p, 1), :] = valEL[k:k + 1, :]

    @pl.when(b == 0)
    def _():
        for c in _copies():
            c.wait()

    odd = [jax.lax.rem(pni_ref[k], 2) == 1 for k in range(_K)]
    rme_all = rme_sc[...].reshape(M * B_, _K * 128)
    blkball = _dotT(selB, rme_all)
    blks = [blkball[:, k * 128:(k + 1) * 128] for k in range(_K)]
    w_rows_C = []
    w_rows_L = []
    for k in range(_K):
        wk = _dotT(fcfl, blks[k])
        w_rows_C.append(jnp.where(odd[k], wk[0:1, N:], wk[0:1, :N]))
        w_rows_L.append(jnp.where(odd[k], wk[1:2, N:], wk[1:2, :N]))
    w8C = jnp.concatenate(w_rows_C, axis=0)
    w8L = jnp.concatenate(w_rows_L, axis=0)

    sc8C = _lrelu(u8C + v_rowC + w8C + cstC)
    sc8L = _lrelu(u8L + v_rowL + w8L + cstL)
    adjb = adj8 > 0.5
    attn8C = _softmax_rows(sc8C, adjb)
    attn8L = _softmax_rows(sc8L, adjb)
    aggC8 = jnp.dot(attn8C, Ht, preferred_element_type=f32)
    aggL8 = jnp.dot(attn8L, Ht, preferred_element_type=f32)
    valH = jnp.where(any8 > 0.5, 0.5 * (aggC8 + aggL8), rowsH8)

    ECr2all = _dotT(blkball, FC)
    ELr2all = _dotT(blkball, FL)
    ECr_sel = jnp.concatenate(
        [jnp.where(odd[k], ECr2all[k * 128 + N:(k + 1) * 128, :],
                   ECr2all[k * 128:k * 128 + N, :]) for k in range(_K)], axis=0)
    ELr_sel = jnp.concatenate(
        [jnp.where(odd[k], ELr2all[k * 128 + N:(k + 1) * 128, :],
                   ELr2all[k * 128:k * 128 + N, :]) for k in range(_K)], axis=0)
    Ht_tiled = jnp.broadcast_to(Ht[None, :, :], (_K, N, D)).reshape(_K * N, D)
    XC = Ht_tiled * ECr_sel
    XL = Ht_tiled * ELr_sel
    bd = (jax.lax.broadcasted_iota(jnp.int32, (_K, _K * N), 1) // N
          == jax.lax.broadcasted_iota(jnp.int32, (_K, _K * N), 0)).astype(f32)
    PC = jnp.tile(attn8C, (1, _K)) * bd
    PL = jnp.tile(attn8L, (1, _K)) * bd
    mkC8 = jnp.dot(PC, XC, preferred_element_type=f32) + WECb * aggC8
    mkL8 = jnp.dot(PL, XL, preferred_element_type=f32) + WELb * aggL8
    mk8 = 0.5 * (mkC8 + mkL8) * any8

    Hn_ref[0] = Ht
    Hm_ref[0] = jnp.zeros((N, D), dtype=f32)
    for k in range(_K):
        i = pni_ref[k]
        Hm_ref[0, pl.ds(i, 1), :] = mk8[k:k + 1, :]
        Hn_ref[0, pl.ds(i, 1), :] = valH[k:k + 1, :]


def kernel(H, E_C, E_L, AH, AE, ME, MH, path_node_indices, path_edge_indices,
           W_H_w, W_H_b, W_EC_w, W_EC_b, W_EL_w, W_EL_b, a_C, a_L, b_C, b_L):
    B, N, ND = H.shape
    M = E_C.shape[1]
    Fe = E_C.shape[2]
    D = W_H_w.shape[1]
    A3 = a_C.shape[0]
    f32 = jnp.float32

    ECtv = jnp.transpose(E_C, (1, 0, 2))
    ELtv = jnp.transpose(E_L, (1, 0, 2))
    AEtv = jnp.transpose(AE, (1, 0, 2))
    MEtv = jnp.transpose(ME, (2, 0, 1))
    maskMM = jnp.asarray((np.arange(M * B)[:, None] // B
                          == np.arange(M)[None, :]).astype(np.float32))
    aCr = jnp.transpose(a_C, (1, 0))
    aLr = jnp.transpose(a_L, (1, 0))
    bCr = jnp.transpose(b_C, (1, 0))
    bLr = jnp.transpose(b_L, (1, 0))

    grid_spec = pltpu.PrefetchScalarGridSpec(
        num_scalar_prefetch=2,
        grid=(B,),
        in_specs=[
            pl.BlockSpec((1, N, ND), lambda b, pni, pei: (b, 0, 0)),
            pl.BlockSpec((M, B, Fe), lambda b, pni, pei: (0, 0, 0)),
            pl.BlockSpec((M, B, Fe), lambda b, pni, pei: (0, 0, 0)),
            pl.BlockSpec((1, N, N), lambda b, pni, pei: (0, 0, 0)),
            pl.BlockSpec((M, B, M), lambda b, pni, pei: (0, 0, 0)),
            pl.BlockSpec(memory_space=pltpu.MemorySpace.HBM),
            pl.BlockSpec((M * B, M), lambda b, pni, pei: (0, 0)),
            pl.BlockSpec((ND, D), lambda b, pni, pei: (0, 0)),
            pl.BlockSpec((D,), lambda b, pni, pei: (0,)),
            pl.BlockSpec((Fe, D), lambda b, pni, pei: (0, 0)),
            pl.BlockSpec((D,), lambda b, pni, pei: (0,)),
            pl.BlockSpec((Fe, D), lambda b, pni, pei: (0, 0)),
            pl.BlockSpec((D,), lambda b, pni, pei: (0,)),
            pl.BlockSpec((1, A3), lambda b, pni, pei: (0, 0)),
            pl.BlockSpec((1, A3), lambda b, pni, pei: (0, 0)),
            pl.BlockSpec((1, A3), lambda b, pni, pei: (0, 0)),
            pl.BlockSpec((1, A3), lambda b, pni, pei: (0, 0)),
        ],
        out_specs=[
            pl.BlockSpec((1, N, D), lambda b, pni, pei: (b, 0, 0)),
            pl.BlockSpec((1, M, D), lambda b, pni, pei: (b, 0, 0)),
            pl.BlockSpec((1, M, D), lambda b, pni, pei: (b, 0, 0)),
            pl.BlockSpec((1, N, D), lambda b, pni, pei: (b, 0, 0)),
        ],
        scratch_shapes=[
            pltpu.VMEM((M, B, _K * 128), f32),
            pltpu.VMEM((N, 9 + N), f32),
            pltpu.VMEM((M, 9 + M), f32),
            pltpu.SemaphoreType.DMA,
        ],
    )

    out_shape = [
        jax.ShapeDtypeStruct((B, N, D), f32),
        jax.ShapeDtypeStruct((B, M, D), f32),
        jax.ShapeDtypeStruct((B, M, D), f32),
        jax.ShapeDtypeStruct((B, N, D), f32),
    ]

    Hn, ECn, ELn, Hm = pl.pallas_call(
        _egat_kernel,
        grid_spec=grid_spec,
        out_shape=out_shape,
        compiler_params=pltpu.CompilerParams(
            dimension_semantics=("arbitrary",),
        ),
    )(path_node_indices, path_edge_indices,
      H, ECtv, ELtv, AH, AEtv, MEtv, maskMM,
      W_H_w, W_H_b, W_EC_w, W_EC_b, W_EL_w, W_EL_b, aCr, aLr, bCr, bLr)

    return (Hn, ECn, ELn, Hm)


# speedup vs baseline: 1.1449x; 1.1449x over previous
import jax
import jax.numpy as jnp
import numpy as np
from jax.experimental import pallas as pl
from jax.experimental.pallas import tpu as pltpu

_NEG = -1e30
_K = 8


def _lrelu(x):
    return jnp.where(x >= 0, x, 0.2 * x)


def _dotT(a, b):
    return jax.lax.dot_general(a, b, (((0,), (0,)), ((), ())),
                               preferred_element_type=jnp.float32)


def _dotNT(a, b):
    return jax.lax.dot_general(a, b, (((1,), (1,)), ((), ())),
                               preferred_element_type=jnp.float32)


def _softmax_rows(score, adj_bool):
    masked = jnp.where(adj_bool, score, _NEG)
    m = jnp.max(masked, axis=1, keepdims=True)
    e = jnp.exp(masked - m)
    return e / jnp.sum(e, axis=1, keepdims=True)


def _egat_kernel(pni_ref, pei_ref,
                 H_ref, ECt_ref, ELt_ref, AH_ref, AEt_ref, MEt_ref, maskMM_ref,
                 WH_ref, WHb_ref, WEC_ref, WECb_ref, WEL_ref, WELb_ref,
                 aCr_ref, aLr_ref, bCr_ref, bLr_ref,
                 Hn_ref, ECn_ref, ELn_ref, Hm_ref,
                 rme_sc, dma_sem):
    N = 64
    M = 50
    D = 256
    B_ = 8
    Fe_ = 128
    b = pl.program_id(0)
    f32 = jnp.float32

    def _copies():
        for k in range(_K):
            icol = pni_ref[k] // 2
            yield pltpu.make_async_copy(
                MEt_ref.at[:, :, pl.ds(icol * 128, 128)],
                rme_sc.at[:, :, pl.ds(k * 128, 128)],
                dma_sem,
            )

    @pl.when(b == 0)
    def _():
        for c in _copies():
            c.start()

    H = H_ref[0]

    onehotB = (jax.lax.broadcasted_iota(jnp.int32, (B_, 1), 0) == b).astype(f32)
    selB = maskMM_ref[...] * jnp.broadcast_to(
        onehotB[None, :, :], (M, B_, 1)).reshape(M * B_, 1)
    ECall = ECt_ref[...].reshape(M * B_, Fe_)
    ELall = ELt_ref[...].reshape(M * B_, Fe_)
    EC = _dotT(selB, ECall)
    EL = _dotT(selB, ELall)
    aCr = aCr_ref[...]
    aLr = aLr_ref[...]
    bCr = bCr_ref[...]
    bLr = bLr_ref[...]
    WECb = WECb_ref[...].reshape(1, D)
    WELb = WELb_ref[...].reshape(1, D)

    Ht = jnp.dot(H, WH_ref[...], preferred_element_type=f32) + WHb_ref[...].reshape(1, D)
    FC = jnp.dot(EC, WEC_ref[...], preferred_element_type=f32)
    FL = jnp.dot(EL, WEL_ref[...], preferred_element_type=f32)

    wvC = _dotNT(WEC_ref[...], aCr[:, 2 * D:])
    wvL = _dotNT(WEL_ref[...], aLr[:, 2 * D:])
    fc = jnp.dot(EC, wvC, preferred_element_type=f32)
    fl = jnp.dot(EL, wvL, preferred_element_type=f32)
    fcfl = jnp.concatenate([fc, fl], axis=1)
    v_rowC = _dotNT(aCr[:, D:2 * D], Ht)
    v_rowL = _dotNT(aLr[:, D:2 * D], Ht)
    cstC = _dotNT(WECb, aCr[:, 2 * D:])
    cstL = _dotNT(WELb, aLr[:, 2 * D:])

    iota = jax.lax.broadcasted_iota(jnp.int32, (N, 1), 0)
    i_row = jnp.concatenate(
        [jnp.full((1, 1), pni_ref[k], jnp.int32) for k in range(_K)], axis=1)
    onehot = (iota == i_row).astype(f32)

    AHf = (AH_ref[0] > 0).astype(f32)
    adj8 = _dotT(onehot, AHf)
    any_adj = jnp.max(AHf, axis=1, keepdims=True)
    any8 = _dotT(onehot, any_adj)
    rowsH8 = _dotT(onehot, Ht)
    u8C = _dotNT(rowsH8, aCr[:, :D])
    u8L = _dotNT(rowsH8, aLr[:, :D])

    ECt = FC + WECb
    ELt = FL + WELb
    y_rowC = _dotNT(bCr[:, D:2 * D], ECt)
    y_rowL = _dotNT(bLr[:, D:2 * D], ELt)
    zC = _dotNT(H[0:1, :], bCr[:, 2 * D:])
    zL = _dotNT(H[0:1, :], bLr[:, 2 * D:])

    iotaE = jax.lax.broadcasted_iota(jnp.int32, (M, 1), 0)
    p_row = jnp.concatenate(
        [jnp.full((1, 1), pei_ref[k], jnp.int32) for k in range(_K)], axis=1)
    onehotE = (iotaE == p_row).astype(f32)
    AEf = (AEt_ref[:, 0, :] > 0).astype(f32)
    adjE8 = _dotT(onehotE, AEf)
    anyE = jnp.max(AEf, axis=1, keepdims=True)
    anyE8 = _dotT(onehotE, anyE)
    rowsEC8 = _dotT(onehotE, ECt)
    rowsEL8 = _dotT(onehotE, ELt)
    x8C = _dotNT(rowsEC8, bCr[:, :D])
    x8L = _dotNT(rowsEL8, bLr[:, :D])

    sc8EC = _lrelu(x8C + y_rowC + zC)
    sc8EL = _lrelu(x8L + y_rowL + zL)
    attn8EC = _softmax_rows(sc8EC, adjE8 > 0.5)
    attn8EL = _softmax_rows(sc8EL, adjE8 > 0.5)
    aggEC8 = jnp.dot(attn8EC, ECt, preferred_element_type=f32)
    aggEL8 = jnp.dot(attn8EL, ELt, preferred_element_type=f32)
    valEC = jnp.where(anyE8 > 0.5, aggEC8, rowsEC8)
    valEL = jnp.where(anyE8 > 0.5, aggEL8, rowsEL8)

    ECn_ref[0] = ECt
    ELn_ref[0] = ELt
    for k in range(_K):
        p = pei_ref[k]
        ECn_ref[0, pl.ds(p, 1), :] = valEC[k:k + 1, :]
        ELn_ref[0, pl.ds(p, 1), :] = valEL[k:k + 1, :]

    @pl.when(b == 0)
    def _():
        for c in _copies():
            c.wait()

    odd = [jax.lax.rem(pni_ref[k], 2) == 1 for k in range(_K)]
    rme_all = rme_sc[...].reshape(M * B_, _K * 128)
    blkball = _dotT(selB, rme_all)
    blks = [blkball[:, k * 128:(k + 1) * 128] for k in range(_K)]
    w_rows_C = []
    w_rows_L = []
    for k in range(_K):
        wk = _dotT(fcfl, blks[k])
        w_rows_C.append(jnp.where(odd[k], wk[0:1, N:], wk[0:1, :N]))
        w_rows_L.append(jnp.where(odd[k], wk[1:2, N:], wk[1:2, :N]))
    w8C = jnp.concatenate(w_rows_C, axis=0)
    w8L = jnp.concatenate(w_rows_L, axis=0)

    sc8C = _lrelu(u8C + v_rowC + w8C + cstC)
    sc8L = _lrelu(u8L + v_rowL + w8L + cstL)
    adjb = adj8 > 0.5
    attn8C = _softmax_rows(sc8C, adjb)
    attn8L = _softmax_rows(sc8L, adjb)
    aggC8 = jnp.dot(attn8C, Ht, preferred_element_type=f32)
    aggL8 = jnp.dot(attn8L, Ht, preferred_element_type=f32)
    valH = jnp.where(any8 > 0.5, 0.5 * (aggC8 + aggL8), rowsH8)

    ECr2all = _dotT(blkball, FC)
    ELr2all = _dotT(blkball, FL)
    ECr_sel = jnp.concatenate(
        [jnp.where(odd[k], ECr2all[k * 128 + N:(k + 1) * 128, :],
                   ECr2all[k * 128:k * 128 + N, :]) for k in range(_K)], axis=0)
    ELr_sel = jnp.concatenate(
        [jnp.where(odd[k], ELr2all[k * 128 + N:(k + 1) * 128, :],
                   ELr2all[k * 128:k * 128 + N, :]) for k in range(_K)], axis=0)
    Ht_tiled = jnp.broadcast_to(Ht[None, :, :], (_K, N, D)).reshape(_K * N, D)
    XC = Ht_tiled * ECr_sel
    XL = Ht_tiled * ELr_sel
    bd = (jax.lax.broadcasted_iota(jnp.int32, (_K, _K * N), 1) // N
          == jax.lax.broadcasted_iota(jnp.int32, (_K, _K * N), 0)).astype(f32)
    PC = jnp.tile(attn8C, (1, _K)) * bd
    PL = jnp.tile(attn8L, (1, _K)) * bd
    mkC8 = jnp.dot(PC, XC, preferred_element_type=f32) + WECb * aggC8
    mkL8 = jnp.dot(PL, XL, preferred_element_type=f32) + WELb * aggL8
    mk8 = 0.5 * (mkC8 + mkL8) * any8

    Hn_ref[0] = Ht
    Hm_ref[0] = jnp.zeros((N, D), dtype=f32)
    for k in range(_K):
        i = pni_ref[k]
        Hm_ref[0, pl.ds(i, 1), :] = mk8[k:k + 1, :]
        Hn_ref[0, pl.ds(i, 1), :] = valH[k:k + 1, :]


def kernel(H, E_C, E_L, AH, AE, ME, MH, path_node_indices, path_edge_indices,
           W_H_w, W_H_b, W_EC_w, W_EC_b, W_EL_w, W_EL_b, a_C, a_L, b_C, b_L):
    B, N, ND = H.shape
    M = E_C.shape[1]
    Fe = E_C.shape[2]
    D = W_H_w.shape[1]
    A3 = a_C.shape[0]
    f32 = jnp.float32

    ECtv = jnp.transpose(E_C, (1, 0, 2))
    ELtv = jnp.transpose(E_L, (1, 0, 2))
    AEtv = jnp.transpose(AE, (1, 0, 2))
    MEtv = jnp.transpose(ME, (2, 0, 1))
    maskMM = jnp.asarray((np.arange(M * B)[:, None] // B
                          == np.arange(M)[None, :]).astype(np.float32))
    aCr = jnp.transpose(a_C, (1, 0))
    aLr = jnp.transpose(a_L, (1, 0))
    bCr = jnp.transpose(b_C, (1, 0))
    bLr = jnp.transpose(b_L, (1, 0))

    grid_spec = pltpu.PrefetchScalarGridSpec(
        num_scalar_prefetch=2,
        grid=(B,),
        in_specs=[
            pl.BlockSpec((1, N, ND), lambda b, pni, pei: (b, 0, 0)),
            pl.BlockSpec((M, B, Fe), lambda b, pni, pei: (0, 0, 0)),
            pl.BlockSpec((M, B, Fe), lambda b, pni, pei: (0, 0, 0)),
            pl.BlockSpec((1, N, N), lambda b, pni, pei: (0, 0, 0)),
            pl.BlockSpec((M, B, M), lambda b, pni, pei: (0, 0, 0)),
            pl.BlockSpec(memory_space=pltpu.MemorySpace.HBM),
            pl.BlockSpec((M * B, M), lambda b, pni, pei: (0, 0)),
            pl.BlockSpec((ND, D), lambda b, pni, pei: (0, 0)),
            pl.BlockSpec((D,), lambda b, pni, pei: (0,)),
            pl.BlockSpec((Fe, D), lambda b, pni, pei: (0, 0)),
            pl.BlockSpec((D,), lambda b, pni, pei: (0,)),
            pl.BlockSpec((Fe, D), lambda b, pni, pei: (0, 0)),
            pl.BlockSpec((D,), lambda b, pni, pei: (0,)),
            pl.BlockSpec((1, A3), lambda b, pni, pei: (0, 0)),
            pl.BlockSpec((1, A3), lambda b, pni, pei: (0, 0)),
            pl.BlockSpec((1, A3), lambda b, pni, pei: (0, 0)),
            pl.BlockSpec((1, A3), lambda b, pni, pei: (0, 0)),
        ],
        out_specs=[
            pl.BlockSpec((1, N, D), lambda b, pni, pei: (b, 0, 0)),
            pl.BlockSpec((1, M, D), lambda b, pni, pei: (b, 0, 0)),
            pl.BlockSpec((1, M, D), lambda b, pni, pei: (b, 0, 0)),
            pl.BlockSpec((1, N, D), lambda b, pni, pei: (b, 0, 0)),
        ],
        scratch_shapes=[
            pltpu.VMEM((M, B, _K * 128), f32),
            pltpu.SemaphoreType.DMA,
        ],
    )

    out_shape = [
        jax.ShapeDtypeStruct((B, N, D), f32),
        jax.ShapeDtypeStruct((B, M, D), f32),
        jax.ShapeDtypeStruct((B, M, D), f32),
        jax.ShapeDtypeStruct((B, N, D), f32),
    ]

    Hn, ECn, ELn, Hm = pl.pallas_call(
        _egat_kernel,
        grid_spec=grid_spec,
        out_shape=out_shape,
        compiler_params=pltpu.CompilerParams(
            dimension_semantics=("arbitrary",),
        ),
    )(path_node_indices, path_edge_indices,
      H, ECtv, ELtv, AH, AEtv, MEtv, maskMM,
      W_H_w, W_H_b, W_EC_w, W_EC_b, W_EL_w, W_EL_b, aCr, aLr, bCr, bLr)

    return (Hn, ECn, ELn, Hm)
